# Initial kernel scaffold; baseline (speedup 1.0000x reference)
import jax
import jax.numpy as jnp
from jax.experimental import pallas as pl


def kernel(x, edge_index, Wv, bv, Wq, bq, Wk, bk):
    raise NotImplementedError("write your pallas kernel here")


# speedup vs baseline: 80.2074x; 80.2074x over previous
import functools

import jax
import jax.numpy as jnp
from jax import lax
from jax.experimental import pallas as pl
from jax.experimental.pallas import tpu as pltpu
from jax.experimental.pallas import tpu_sc as plsc


def _qk_body(x_ref, wv_ref, wqk_ref, bv_ref, bqk_ref, out_ref):
    w2 = jnp.dot(wv_ref[...], wqk_ref[...], preferred_element_type=jnp.float32)
    b2 = lax.dot_general(
        wqk_ref[...], bv_ref[...], (((0,), (1,)), ((), ())),
        preferred_element_type=jnp.float32) + bqk_ref[...]
    out_ref[...] = lax.dot_general(
        w2, x_ref[...], (((0,), (1,)), ((), ())),
        preferred_element_type=jnp.float32) + b2


def _make_edge_kernel(n_nodes, n_edges, edges_per_worker):
    mesh = plsc.VectorSubcoreMesh(core_axis_name="c", subcore_axis_name="s")
    info = plsc.get_sparse_core_info()
    num_cores = info.num_cores

    @functools.partial(
        pl.kernel,
        mesh=mesh,
        out_type=jax.ShapeDtypeStruct((n_edges,), jnp.float32),
        compiler_params=pltpu.CompilerParams(
            needs_layout_passes=False,
            use_tc_tiling_on_sc=False,
        ),
        scratch_types=[
            pltpu.VMEM((n_nodes,), jnp.float32),
            pltpu.VMEM((n_nodes,), jnp.float32),
            pltpu.VMEM((edges_per_worker,), jnp.int32),
            pltpu.VMEM((edges_per_worker,), jnp.int32),
            pltpu.VMEM((edges_per_worker,), jnp.float32),
        ],
    )
    def edge_kernel(qk_hbm, src_hbm, dst_hbm, out_hbm,
                    q_v, k_v, src_v, dst_v, out_v):
        wid = lax.axis_index("s") * num_cores + lax.axis_index("c")
        base = wid * edges_per_worker
        pltpu.sync_copy(qk_hbm.at[0], q_v)
        pltpu.sync_copy(qk_hbm.at[1], k_v)
        pltpu.sync_copy(src_hbm.at[pl.ds(base, edges_per_worker)], src_v)
        pltpu.sync_copy(dst_hbm.at[pl.ds(base, edges_per_worker)], dst_v)

        def body(i, carry):
            off = i * 16
            s_idx = src_v[pl.ds(off, 16)]
            d_idx = dst_v[pl.ds(off, 16)]
            e = plsc.load_gather(q_v, [s_idx]) + plsc.load_gather(k_v, [d_idx])
            out_v[pl.ds(off, 16)] = jnp.where(e > 0, e, 0.2 * e)
            return carry

        lax.fori_loop(0, edges_per_worker // 16, body, 0)
        pltpu.sync_copy(out_v, out_hbm.at[pl.ds(base, edges_per_worker)])

    return edge_kernel


def kernel(x, edge_index, Wv, bv, Wq, bq, Wk, bk):
    n, d_in = x.shape
    e = edge_index.shape[1]

    wqk = jnp.concatenate(
        [Wq, Wk, jnp.zeros((Wq.shape[0], 6), jnp.float32)], axis=1)
    bqk = jnp.concatenate(
        [bq, bk, jnp.zeros((6,), jnp.float32)]).reshape(8, 1)
    bv2d = bv.reshape(1, d_in)

    qk = pl.pallas_call(
        _qk_body,
        out_shape=jax.ShapeDtypeStruct((8, n), jnp.float32),
    )(x, Wv, wqk, bv2d, bqk)

    ei = edge_index.astype(jnp.int32)
    src = ei[0]
    dst = ei[1]

    epw = e // 32
    edge_kernel = _make_edge_kernel(n, e, epw)
    coeff = edge_kernel(qk, src, dst)
    return coeff.reshape(e, 1)


# speedup vs baseline: 119.4706x; 1.4895x over previous
import functools

import jax
import jax.numpy as jnp
from jax import lax
from jax.experimental import pallas as pl
from jax.experimental.pallas import tpu as pltpu
from jax.experimental.pallas import tpu_sc as plsc


def _qk_body(x_ref, wv_ref, wqk_ref, bv_ref, bqk_ref, out_ref):
    w2 = jnp.dot(wv_ref[...], wqk_ref[...], preferred_element_type=jnp.float32)
    b2 = lax.dot_general(
        wqk_ref[...], bv_ref[...], (((0,), (1,)), ((), ())),
        preferred_element_type=jnp.float32) + bqk_ref[...]
    n = x_ref.shape[0]
    out_ref[:, :n] = lax.dot_general(
        w2, x_ref[...], (((0,), (1,)), ((), ())),
        preferred_element_type=jnp.float32) + b2


def _make_edge_kernel(n_tiles, n_edges, n_chunks, chunks_per_worker):
    mesh = plsc.VectorSubcoreMesh(core_axis_name="c", subcore_axis_name="s")
    info = plsc.get_sparse_core_info()
    num_cores = info.num_cores
    epw = chunks_per_worker * 128

    @functools.partial(
        pl.kernel,
        mesh=mesh,
        out_type=jax.ShapeDtypeStruct((1, n_edges), jnp.float32),
        compiler_params=pltpu.CompilerParams(
            needs_layout_passes=False,
            use_tc_tiling_on_sc=False,
        ),
        scratch_types=[
            pltpu.VMEM((n_tiles, 128), jnp.float32),
            pltpu.VMEM((n_tiles, 128), jnp.float32),
            pltpu.VMEM((chunks_per_worker, 2, 128), jnp.int32),
            pltpu.VMEM((epw,), jnp.float32),
            pltpu.SemaphoreType.DMA,
        ],
    )
    def edge_kernel(qk_hbm, ei_hbm, out_hbm, q_v, k_v, ei_v, out_v, sem):
        wid = lax.axis_index("s") * num_cores + lax.axis_index("c")
        base_c = jnp.minimum(
            wid * chunks_per_worker, n_chunks - chunks_per_worker)
        cp_q = pltpu.async_copy(qk_hbm.at[:, 0, :], q_v, sem)
        cp_k = pltpu.async_copy(qk_hbm.at[:, 1, :], k_v, sem)
        cp_e = pltpu.async_copy(
            ei_hbm.at[pl.ds(base_c, chunks_per_worker)], ei_v, sem)
        cp_q.wait()
        cp_k.wait()
        cp_e.wait()

        @plsc.parallel_loop(0, epw, step=16, unroll=8)
        def _(off):
            c = off >> 7
            p = off & 127
            s_idx = ei_v[c, 0, pl.ds(p, 16)]
            d_idx = ei_v[c, 1, pl.ds(p, 16)]
            e = (plsc.load_gather(q_v, [s_idx >> 7, s_idx & 127])
                 + plsc.load_gather(k_v, [d_idx >> 7, d_idx & 127]))
            out_v[pl.ds(off, 16)] = jnp.where(e > 0, e, 0.2 * e)

        pltpu.sync_copy(
            out_v, out_hbm.at[0, pl.ds(base_c * 128, epw)])

    return edge_kernel


def kernel(x, edge_index, Wv, bv, Wq, bq, Wk, bk):
    n, d_in = x.shape
    e = edge_index.shape[1]
    n_tiles = (n + 127) // 128
    np_ = n_tiles * 128

    wqk = jnp.concatenate(
        [Wq, Wk, jnp.zeros((Wq.shape[0], 6), jnp.float32)], axis=1)
    bqk = jnp.concatenate(
        [bq, bk, jnp.zeros((6,), jnp.float32)]).reshape(8, 1)
    bv2d = bv.reshape(1, d_in)

    qk = pl.pallas_call(
        _qk_body,
        out_shape=jax.ShapeDtypeStruct((8, np_), jnp.float32),
    )(x, Wv, wqk, bv2d, bqk)
    qk3 = qk.reshape(8, n_tiles, 128).transpose(1, 0, 2)

    ei = edge_index.astype(jnp.int32)
    n_chunks = e // 128
    ei3 = ei.reshape(2, n_chunks, 128).transpose(1, 0, 2)

    n_workers = 32
    cpw = (n_chunks + n_workers - 1) // n_workers
    edge_kernel = _make_edge_kernel(n_tiles, e, n_chunks, cpw)
    return edge_kernel(qk3, ei3).T
